# Initial kernel scaffold; baseline (speedup 1.0000x reference)
import jax
import jax.numpy as jnp
from jax.experimental import pallas as pl


def kernel(x, edge_index, edge_weight, params):
    raise NotImplementedError("write your pallas kernel here")


# speedup vs baseline: 8.1214x; 8.1214x over previous
import functools

import jax
import jax.numpy as jnp
from jax import lax
from jax.experimental import pallas as pl
from jax.experimental.pallas import tpu as pltpu
from jax.experimental.pallas import tpu_sc as plsc

N = 10000
E = 320000
D_IN = 128
D = 64
OUT_LEN = 12
FE = 4

NPAD = 10240
RPT = 640
CHUNK = 80
EROWS = E // CHUNK
NW = 32
CPT = EROWS // NW
BLK = 1000
GRID = N // BLK


def _sigmoid(x):
    return 1.0 / (1.0 + jnp.exp(-x))


def _ln(x, g, b):
    m = jnp.mean(x, axis=-1, keepdims=True)
    v = jnp.mean((x - m) ** 2, axis=-1, keepdims=True)
    return (x - m) / jnp.sqrt(v + 1e-5) * g + b


def _pre_body(x_ref, we, be, wp, bp, wq, bq, wr, br, xe_ref, t1_ref):
    xe = jnp.dot(x_ref[...], we[...], preferred_element_type=jnp.float32) + be[...]
    p = jnp.dot(xe, wp[...], preferred_element_type=jnp.float32) + bp[...]
    q = jnp.dot(xe, wq[...], preferred_element_type=jnp.float32) + bq[...]
    r = jnp.dot(xe, wr[...], preferred_element_type=jnp.float32) + br[...]
    xe_ref[...] = xe
    t1_ref[...] = jnp.maximum(p * _sigmoid(q) + r, 0.0)


def _pre_call(x, we, be, wp, bp, wq, bq, wr, br):
    full = lambda s: pl.BlockSpec(s, lambda i: (0, 0))
    return pl.pallas_call(
        _pre_body,
        grid=(GRID,),
        in_specs=[
            pl.BlockSpec((BLK, D_IN), lambda i: (i, 0)),
            full((D_IN, D)), full((1, D)),
            full((D, D)), full((1, D)),
            full((D, D)), full((1, D)),
            full((D, D)), full((1, D)),
        ],
        out_specs=[
            pl.BlockSpec((BLK, D), lambda i: (i, 0)),
            pl.BlockSpec((BLK, D), lambda i: (i, 0)),
        ],
        out_shape=[
            jax.ShapeDtypeStruct((N, D), jnp.float32),
            jax.ShapeDtypeStruct((N, D), jnp.float32),
        ],
    )(x, we, be, wp, bp, wq, bq, wr, br)


def _sc_body(t1_hbm, src_hbm, dst_hbm, w_hbm, agg_out, deg_out,
             t1_sh, agg_sh, deg_sh, srcc, dstc, wc, rows, stage, zd):
    cid = lax.axis_index("c")
    sid = lax.axis_index("s")
    wid = cid * 16 + sid
    zero16 = jnp.zeros((16,), jnp.float32)
    base_r = sid * RPT

    def _zrow(i, _):
        for j in range(4):
            stage[i, pl.ds(j * 16, 16)] = zero16
        return 0
    lax.fori_loop(0, RPT, _zrow, 0)

    def _zd(i, _):
        zd[pl.ds(i * 16, 16)] = zero16
        return 0
    lax.fori_loop(0, RPT // 16, _zd, 0)

    pltpu.sync_copy(stage, agg_sh.at[pl.ds(base_r, RPT)])
    pltpu.sync_copy(zd, deg_sh.at[pl.ds(base_r, RPT)])

    @pl.when(sid < 15)
    def _():
        pltpu.sync_copy(t1_hbm.at[pl.ds(base_r, RPT)], stage)
        pltpu.sync_copy(stage, t1_sh.at[pl.ds(base_r, RPT)])

    @pl.when(sid == 15)
    def _():
        pltpu.sync_copy(t1_hbm.at[pl.ds(9600, 400)], stage.at[pl.ds(0, 400)])
        pltpu.sync_copy(stage.at[pl.ds(0, 400)], t1_sh.at[pl.ds(9600, 400)])

    plsc.subcore_barrier()

    ebase = wid * CPT

    def _edge_chunk(c, _):
        r = ebase + c
        pltpu.sync_copy(src_hbm.at[r], srcc)
        pltpu.sync_copy(dst_hbm.at[r], dstc)
        pltpu.sync_copy(w_hbm.at[r], wc)
        pltpu.sync_copy(t1_sh.at[srcc], rows)

        def _scale(k, _):
            wv = wc[pl.ds(k * 16, 16)]
            for j in range(16):
                ws = wv[j]
                row = k * 16 + j
                for q in range(4):
                    sl = pl.ds(q * 16, 16)
                    rows[row, sl] = rows[row, sl] * ws
            return 0
        lax.fori_loop(0, CHUNK // 16, _scale, 0)

        pltpu.sync_copy(wc, deg_sh.at[dstc], add=True)
        pltpu.sync_copy(rows, agg_sh.at[dstc], add=True)
        return 0
    lax.fori_loop(0, CPT, _edge_chunk, 0)

    plsc.subcore_barrier()

    pltpu.sync_copy(agg_sh.at[pl.ds(base_r, RPT)], stage)
    pltpu.sync_copy(stage, agg_out.at[pl.ds(cid * NPAD + base_r, RPT)])
    pltpu.sync_copy(deg_sh.at[pl.ds(base_r, RPT)], zd)
    pltpu.sync_copy(zd, deg_out.at[pl.ds(cid * NPAD + base_r, RPT)])


@functools.lru_cache(maxsize=1)
def _make_sc_call():
    return functools.partial(
        pl.kernel,
        out_type=[
            jax.ShapeDtypeStruct((2 * NPAD, D), jnp.float32),
            jax.ShapeDtypeStruct((2 * NPAD,), jnp.float32),
        ],
        mesh=plsc.VectorSubcoreMesh(core_axis_name="c", subcore_axis_name="s",
                                    num_cores=2, num_subcores=16),
        compiler_params=pltpu.CompilerParams(use_tc_tiling_on_sc=False),
        scratch_types=[
            pltpu.VMEM_SHARED((NPAD, D), jnp.float32),
            pltpu.VMEM_SHARED((NPAD, D), jnp.float32),
            pltpu.VMEM_SHARED((NPAD,), jnp.float32),
            pltpu.VMEM((CHUNK,), jnp.int32),
            pltpu.VMEM((CHUNK,), jnp.int32),
            pltpu.VMEM((CHUNK,), jnp.float32),
            pltpu.VMEM((CHUNK, D), jnp.float32),
            pltpu.VMEM((RPT, D), jnp.float32),
            pltpu.VMEM((RPT,), jnp.float32),
        ],
    )(_sc_body)


def _sc_call(t1, src2, dst2, w2):
    return _make_sc_call()(t1, src2, dst2, w2)


def _post_body(xe_ref, a0, a1, d0, d1,
               wc, bc, wp2, bp2, wq2, bq2, wr2, br2,
               g1, b1, g2, b2, wf1, bf1, wf2, bf2,
               wfs, bfs, wfgq, wfgo, bfg, gb1, bb1, wo, bo,
               out_ref):
    xe = xe_ref[...]
    agg = (a0[...] + a1[...]) * (1.0 / (d0[...] + d1[...] + 1e-5))
    t2 = jnp.maximum(jnp.dot(agg, wc[...], preferred_element_type=jnp.float32) + bc[...], 0.0)
    p2 = jnp.dot(t2, wp2[...], preferred_element_type=jnp.float32) + bp2[...]
    q2 = jnp.dot(t2, wq2[...], preferred_element_type=jnp.float32) + bq2[...]
    r2 = jnp.dot(t2, wr2[...], preferred_element_type=jnp.float32) + br2[...]
    o = jnp.maximum(p2 * _sigmoid(q2) + r2, 0.0)
    xn = _ln(xe, g1[...], b1[...])
    ff = jnp.maximum(jnp.dot(xn, wf1[...], preferred_element_type=jnp.float32) + bf1[...], 0.0)
    ff = jnp.dot(ff, wf2[...], preferred_element_type=jnp.float32) + bf2[...]
    us = _ln(ff + xn, g2[...], b2[...])
    fgx = (jnp.dot(xe, wfgq[...], preferred_element_type=jnp.float32)
           + jnp.dot(o, wfgo[...], preferred_element_type=jnp.float32) + bfg[...])
    g = _sigmoid(jnp.dot(us, wfs[...], preferred_element_type=jnp.float32) + bfs[...] + fgx)
    st = g * us + (1.0 - g) * fgx
    x1 = _ln(st + xe, gb1[...], bb1[...])
    out_ref[...] = jnp.dot(x1, wo[...], preferred_element_type=jnp.float32) + bo[...]


def _post_call(xe, a0, a1, d0, d1, *ws):
    full = lambda a: pl.BlockSpec(a.shape, lambda i: (0,) * a.ndim)
    blk = lambda: pl.BlockSpec((BLK, D), lambda i: (i, 0))
    return pl.pallas_call(
        _post_body,
        grid=(GRID,),
        in_specs=[blk(), blk(), blk(),
                  pl.BlockSpec((BLK, 1), lambda i: (i, 0)),
                  pl.BlockSpec((BLK, 1), lambda i: (i, 0))]
                 + [full(w) for w in ws],
        out_specs=pl.BlockSpec((BLK, OUT_LEN), lambda i: (i, 0)),
        out_shape=jax.ShapeDtypeStruct((N, OUT_LEN), jnp.float32),
    )(xe, a0, a1, d0, d1, *ws)


def kernel(x, edge_index, edge_weight, params):
    p = params
    r1 = lambda v: v.reshape(1, -1)
    src2 = edge_index[0].reshape(EROWS, CHUNK)
    dst2 = edge_index[1].reshape(EROWS, CHUNK)
    w2 = edge_weight.reshape(EROWS, CHUNK)

    xe, t1 = _pre_call(x, p['W_embed'], r1(p['b_embed']),
                       p['Wp1'], r1(p['bp1']), p['Wq1'], r1(p['bq1']),
                       p['Wr1'], r1(p['br1']))

    aggp, degp = _sc_call(t1, src2, dst2, w2)
    a0 = aggp[:N]
    a1 = aggp[NPAD:NPAD + N]
    d0 = degp[:N].reshape(N, 1)
    d1 = degp[NPAD:NPAD + N].reshape(N, 1)

    return _post_call(
        xe, a0, a1, d0, d1,
        p['Wc'], r1(p['bc']),
        p['Wp2'], r1(p['bp2']), p['Wq2'], r1(p['bq2']), p['Wr2'], r1(p['br2']),
        r1(p['g1']), r1(p['b1']), r1(p['g2']), r1(p['b2']),
        p['Wf1'], r1(p['bf1']), p['Wf2'], r1(p['bf2']),
        p['Wfs'], r1(p['bfs']),
        p['Wfg'][:D], p['Wfg'][D:], r1(p['bfg']),
        r1(p['gb1']), r1(p['bb1']),
        p['W_out'], r1(p['b_out']),
    )


# speedup vs baseline: 13.3078x; 1.6386x over previous
import functools

import jax
import jax.numpy as jnp
from jax import lax
from jax.experimental import pallas as pl
from jax.experimental.pallas import tpu as pltpu
from jax.experimental.pallas import tpu_sc as plsc

N = 10000
E = 320000
D_IN = 128
D = 64
OUT_LEN = 12
FE = 4

NPAD = 10240
RPT = 640
CHUNK = 128
EPAD = 327680
EROWS = EPAD // CHUNK
NW = 32
CPT = EROWS // NW
NBUF = 4
BLK = 1000
GRID = N // BLK


def _sigmoid(x):
    return 1.0 / (1.0 + jnp.exp(-x))


def _ln(x, g, b):
    m = jnp.mean(x, axis=-1, keepdims=True)
    v = jnp.mean((x - m) ** 2, axis=-1, keepdims=True)
    return (x - m) / jnp.sqrt(v + 1e-5) * g + b


def _pre_body(x_ref, we, be, wp, bp, wq, bq, wr, br, xe_ref, t1_ref):
    xe = jnp.dot(x_ref[...], we[...], preferred_element_type=jnp.float32) + be[...]
    p = jnp.dot(xe, wp[...], preferred_element_type=jnp.float32) + bp[...]
    q = jnp.dot(xe, wq[...], preferred_element_type=jnp.float32) + bq[...]
    r = jnp.dot(xe, wr[...], preferred_element_type=jnp.float32) + br[...]
    xe_ref[...] = xe
    t1_ref[...] = jnp.maximum(p * _sigmoid(q) + r, 0.0)


def _pre_call(x, we, be, wp, bp, wq, bq, wr, br):
    full = lambda s: pl.BlockSpec(s, lambda i: (0, 0))
    return pl.pallas_call(
        _pre_body,
        grid=(GRID,),
        in_specs=[
            pl.BlockSpec((BLK, D_IN), lambda i: (i, 0)),
            full((D_IN, D)), full((1, D)),
            full((D, D)), full((1, D)),
            full((D, D)), full((1, D)),
            full((D, D)), full((1, D)),
        ],
        out_specs=[
            pl.BlockSpec((BLK, D), lambda i: (i, 0)),
            pl.BlockSpec((BLK, D), lambda i: (i, 0)),
        ],
        out_shape=[
            jax.ShapeDtypeStruct((N, D), jnp.float32),
            jax.ShapeDtypeStruct((N, D), jnp.float32),
        ],
    )(x, we, be, wp, bp, wq, bq, wr, br)


def _sc_body(t1_hbm, src_hbm, dst_hbm, w_hbm, agg_out, deg_out,
             agg_sh, deg_sh, src_all, dst_all, w_all,
             r0, r1, r2, r3, zb, zd,
             g0, g1, g2, g3, s0, s1, s2, s3, dsem):
    cid = lax.axis_index("c")
    sid = lax.axis_index("s")
    wid = cid * 16 + sid
    zero16 = jnp.zeros((16,), jnp.float32)
    base_r = sid * RPT
    rows = (r0, r1, r2, r3)
    gsems = (g0, g1, g2, g3)
    ssems = (s0, s1, s2, s3)

    def _zrow(i, _):
        for j in range(4):
            zb[i, pl.ds(j * 16, 16)] = zero16
        return 0
    lax.fori_loop(0, RPT // 10, _zrow, 0)

    def _zdl(i, _):
        zd[pl.ds(i * 16, 16)] = zero16
        return 0
    lax.fori_loop(0, RPT // 16, _zdl, 0)

    for part in range(10):
        pltpu.sync_copy(zb, agg_sh.at[pl.ds(base_r + part * (RPT // 10), RPT // 10)])
    pltpu.sync_copy(zd, deg_sh.at[pl.ds(base_r, RPT)])

    ebase = wid * CPT
    pltpu.sync_copy(src_hbm.at[pl.ds(ebase, CPT)], src_all)
    pltpu.sync_copy(dst_hbm.at[pl.ds(ebase, CPT)], dst_all)
    pltpu.sync_copy(w_hbm.at[pl.ds(ebase, CPT)], w_all)

    plsc.subcore_barrier()

    def _gather(c, k):
        return pltpu.make_async_copy(t1_hbm.at[src_all.at[c]], rows[k], gsems[k])

    def _scatter(c, k):
        return pltpu.make_async_copy(rows[k], agg_sh.at[dst_all.at[c]], ssems[k])

    def _degsc(c):
        return pltpu.make_async_copy(w_all.at[c], deg_sh.at[dst_all.at[c]], dsem)

    for k in range(NBUF):
        _gather(k, k).start()

    def _iter(p, _):
        for k in range(NBUF):
            c = p * NBUF + k
            _gather(c, k).wait()

            def _scale(q, _):
                wv = w_all[c, pl.ds(q * 16, 16)]
                for j in range(16):
                    ws = wv[j]
                    row = q * 16 + j
                    for t in range(4):
                        sl = pl.ds(t * 16, 16)
                        rows[k][row, sl] = rows[k][row, sl] * ws
                return 0
            lax.fori_loop(0, CHUNK // 16, _scale, 0)

            _scatter(c, k).start(add=True)
            _degsc(c).start(add=True)

            @pl.when(p < CPT // NBUF - 1)
            def _():
                _scatter(c, k).wait()
                _degsc(c).wait()
                _gather(c + NBUF, k).start()
        return 0
    lax.fori_loop(0, CPT // NBUF, _iter, 0)

    for k in range(NBUF):
        c = CPT - NBUF + k
        _scatter(c, k).wait()
        _degsc(c).wait()

    plsc.subcore_barrier()

    pltpu.sync_copy(agg_sh.at[pl.ds(base_r, RPT)],
                    agg_out.at[pl.ds(cid * NPAD + base_r, RPT)])
    pltpu.sync_copy(deg_sh.at[pl.ds(base_r, RPT)],
                    deg_out.at[pl.ds(cid * NPAD + base_r, RPT)])


@functools.lru_cache(maxsize=1)
def _make_sc_call():
    return functools.partial(
        pl.kernel,
        out_type=[
            jax.ShapeDtypeStruct((2 * NPAD, D), jnp.float32),
            jax.ShapeDtypeStruct((2 * NPAD,), jnp.float32),
        ],
        mesh=plsc.VectorSubcoreMesh(core_axis_name="c", subcore_axis_name="s",
                                    num_cores=2, num_subcores=16),
        compiler_params=pltpu.CompilerParams(use_tc_tiling_on_sc=False),
        scratch_types=[
            pltpu.VMEM_SHARED((NPAD, D), jnp.float32),
            pltpu.VMEM_SHARED((NPAD,), jnp.float32),
            pltpu.VMEM((CPT, CHUNK), jnp.int32),
            pltpu.VMEM((CPT, CHUNK), jnp.int32),
            pltpu.VMEM((CPT, CHUNK), jnp.float32),
            pltpu.VMEM((CHUNK, D), jnp.float32),
            pltpu.VMEM((CHUNK, D), jnp.float32),
            pltpu.VMEM((CHUNK, D), jnp.float32),
            pltpu.VMEM((CHUNK, D), jnp.float32),
            pltpu.VMEM((RPT // 10, D), jnp.float32),
            pltpu.VMEM((RPT,), jnp.float32),
            pltpu.SemaphoreType.DMA,
            pltpu.SemaphoreType.DMA,
            pltpu.SemaphoreType.DMA,
            pltpu.SemaphoreType.DMA,
            pltpu.SemaphoreType.DMA,
            pltpu.SemaphoreType.DMA,
            pltpu.SemaphoreType.DMA,
            pltpu.SemaphoreType.DMA,
            pltpu.SemaphoreType.DMA,
        ],
    )(_sc_body)


def _sc_call(t1, src2, dst2, w2):
    return _make_sc_call()(t1, src2, dst2, w2)


def _post_body(xe_ref, a0, a1, d0, d1,
               wc, bc, wp2, bp2, wq2, bq2, wr2, br2,
               g1, b1, g2, b2, wf1, bf1, wf2, bf2,
               wfs, bfs, wfgq, wfgo, bfg, gb1, bb1, wo, bo,
               out_ref):
    xe = xe_ref[...]
    agg = (a0[...] + a1[...]) * (1.0 / (d0[...] + d1[...] + 1e-5))
    t2 = jnp.maximum(jnp.dot(agg, wc[...], preferred_element_type=jnp.float32) + bc[...], 0.0)
    p2 = jnp.dot(t2, wp2[...], preferred_element_type=jnp.float32) + bp2[...]
    q2 = jnp.dot(t2, wq2[...], preferred_element_type=jnp.float32) + bq2[...]
    r2 = jnp.dot(t2, wr2[...], preferred_element_type=jnp.float32) + br2[...]
    o = jnp.maximum(p2 * _sigmoid(q2) + r2, 0.0)
    xn = _ln(xe, g1[...], b1[...])
    ff = jnp.maximum(jnp.dot(xn, wf1[...], preferred_element_type=jnp.float32) + bf1[...], 0.0)
    ff = jnp.dot(ff, wf2[...], preferred_element_type=jnp.float32) + bf2[...]
    us = _ln(ff + xn, g2[...], b2[...])
    fgx = (jnp.dot(xe, wfgq[...], preferred_element_type=jnp.float32)
           + jnp.dot(o, wfgo[...], preferred_element_type=jnp.float32) + bfg[...])
    g = _sigmoid(jnp.dot(us, wfs[...], preferred_element_type=jnp.float32) + bfs[...] + fgx)
    st = g * us + (1.0 - g) * fgx
    x1 = _ln(st + xe, gb1[...], bb1[...])
    out_ref[...] = jnp.dot(x1, wo[...], preferred_element_type=jnp.float32) + bo[...]


def _post_call(xe, a0, a1, d0, d1, *ws):
    full = lambda a: pl.BlockSpec(a.shape, lambda i: (0,) * a.ndim)
    blk = lambda: pl.BlockSpec((BLK, D), lambda i: (i, 0))
    return pl.pallas_call(
        _post_body,
        grid=(GRID,),
        in_specs=[blk(), blk(), blk(),
                  pl.BlockSpec((BLK, 1), lambda i: (i, 0)),
                  pl.BlockSpec((BLK, 1), lambda i: (i, 0))]
                 + [full(w) for w in ws],
        out_specs=pl.BlockSpec((BLK, OUT_LEN), lambda i: (i, 0)),
        out_shape=jax.ShapeDtypeStruct((N, OUT_LEN), jnp.float32),
    )(xe, a0, a1, d0, d1, *ws)


def kernel(x, edge_index, edge_weight, params):
    p = params
    r1 = lambda v: v.reshape(1, -1)
    npad = EPAD - E
    idx_pad = (jnp.arange(npad, dtype=jnp.int32) * 13) % N
    src2 = jnp.concatenate([edge_index[0], idx_pad]).reshape(EROWS, CHUNK)
    dst2 = jnp.concatenate([edge_index[1], idx_pad]).reshape(EROWS, CHUNK)
    w2 = jnp.concatenate([edge_weight,
                          jnp.zeros((npad,), jnp.float32)]).reshape(EROWS, CHUNK)

    xe, t1 = _pre_call(x, p['W_embed'], r1(p['b_embed']),
                       p['Wp1'], r1(p['bp1']), p['Wq1'], r1(p['bq1']),
                       p['Wr1'], r1(p['br1']))

    aggp, degp = _sc_call(t1, src2, dst2, w2)
    a0 = aggp[:N]
    a1 = aggp[NPAD:NPAD + N]
    d0 = degp[:N].reshape(N, 1)
    d1 = degp[NPAD:NPAD + N].reshape(N, 1)

    return _post_call(
        xe, a0, a1, d0, d1,
        p['Wc'], r1(p['bc']),
        p['Wp2'], r1(p['bp2']), p['Wq2'], r1(p['bq2']), p['Wr2'], r1(p['br2']),
        r1(p['g1']), r1(p['b1']), r1(p['g2']), r1(p['b2']),
        p['Wf1'], r1(p['bf1']), p['Wf2'], r1(p['bf2']),
        p['Wfs'], r1(p['bfs']),
        p['Wfg'][:D], p['Wfg'][D:], r1(p['bfg']),
        r1(p['gb1']), r1(p['bb1']),
        p['W_out'], r1(p['b_out']),
    )


# speedup vs baseline: 15.3164x; 1.1509x over previous
import functools

import jax
import jax.numpy as jnp
from jax import lax
from jax.experimental import pallas as pl
from jax.experimental.pallas import tpu as pltpu
from jax.experimental.pallas import tpu_sc as plsc

N = 10000
E = 320000
D_IN = 128
D = 64
OUT_LEN = 12
FE = 4

NPAD = 10240
RPT = 640
CHUNK = 128
EPAD = 327680
EROWS = EPAD // CHUNK
NW = 32
CPT = EROWS // NW
NBUF = 4
BLK = 1000
GRID = N // BLK


def _sigmoid(x):
    return 1.0 / (1.0 + jnp.exp(-x))


def _ln(x, g, b):
    m = jnp.mean(x, axis=-1, keepdims=True)
    v = jnp.mean((x - m) ** 2, axis=-1, keepdims=True)
    return (x - m) / jnp.sqrt(v + 1e-5) * g + b


def _pre_body(x_ref, we, be, wp, bp, wq, bq, wr, br, xe_ref, t1_ref):
    xe = jnp.dot(x_ref[...], we[...], preferred_element_type=jnp.float32) + be[...]
    p = jnp.dot(xe, wp[...], preferred_element_type=jnp.float32) + bp[...]
    q = jnp.dot(xe, wq[...], preferred_element_type=jnp.float32) + bq[...]
    r = jnp.dot(xe, wr[...], preferred_element_type=jnp.float32) + br[...]
    xe_ref[...] = xe
    t1_ref[...] = jnp.maximum(p * _sigmoid(q) + r, 0.0)


def _pre_call(x, we, be, wp, bp, wq, bq, wr, br):
    full = lambda s: pl.BlockSpec(s, lambda i: (0, 0))
    return pl.pallas_call(
        _pre_body,
        grid=(GRID,),
        in_specs=[
            pl.BlockSpec((BLK, D_IN), lambda i: (i, 0)),
            full((D_IN, D)), full((1, D)),
            full((D, D)), full((1, D)),
            full((D, D)), full((1, D)),
            full((D, D)), full((1, D)),
        ],
        out_specs=[
            pl.BlockSpec((BLK, D), lambda i: (i, 0)),
            pl.BlockSpec((BLK, D), lambda i: (i, 0)),
        ],
        out_shape=[
            jax.ShapeDtypeStruct((N, D), jnp.float32),
            jax.ShapeDtypeStruct((N, D), jnp.float32),
        ],
    )(x, we, be, wp, bp, wq, bq, wr, br)


def _sc_body(t1_hbm, src_hbm, dst_hbm, w_hbm, agg0_out, agg1_out,
             deg0_out, deg1_out,
             agg_sh, deg_sh, src_all, dst_all, w_all,
             r0, r1, r2, r3, zb, zd,
             g0, g1, g2, g3, s0, s1, s2, s3, dsem):
    cid = lax.axis_index("c")
    sid = lax.axis_index("s")
    wid = cid * 16 + sid
    zero16 = jnp.zeros((16,), jnp.float32)
    base_r = sid * RPT
    rows = (r0, r1, r2, r3)
    gsems = (g0, g1, g2, g3)
    ssems = (s0, s1, s2, s3)

    def _zrow(i, _):
        for j in range(4):
            zb[i, pl.ds(j * 16, 16)] = zero16
        return 0
    lax.fori_loop(0, RPT // 10, _zrow, 0)

    def _zdl(i, _):
        zd[pl.ds(i * 16, 16)] = zero16
        return 0
    lax.fori_loop(0, RPT // 16, _zdl, 0)

    for part in range(10):
        pltpu.sync_copy(zb, agg_sh.at[pl.ds(base_r + part * (RPT // 10), RPT // 10)])
    pltpu.sync_copy(zd, deg_sh.at[pl.ds(base_r, RPT)])

    ebase = wid * CPT
    pltpu.sync_copy(src_hbm.at[pl.ds(ebase, CPT)], src_all)
    pltpu.sync_copy(dst_hbm.at[pl.ds(ebase, CPT)], dst_all)
    pltpu.sync_copy(w_hbm.at[pl.ds(ebase, CPT)], w_all)

    plsc.subcore_barrier()

    def _gather(c, k):
        return pltpu.make_async_copy(t1_hbm.at[src_all.at[c]], rows[k], gsems[k])

    def _scatter(c, k):
        return pltpu.make_async_copy(rows[k], agg_sh.at[dst_all.at[c]], ssems[k])

    def _degsc(c):
        return pltpu.make_async_copy(w_all.at[c], deg_sh.at[dst_all.at[c]], dsem)

    for k in range(NBUF):
        _gather(k, k).start()

    def _iter(p, _):
        for k in range(NBUF):
            c = p * NBUF + k
            _gather(c, k).wait()

            def _scale(q, _):
                wv = w_all[c, pl.ds(q * 16, 16)]
                for j in range(16):
                    ws = wv[j]
                    row = q * 16 + j
                    for t in range(4):
                        sl = pl.ds(t * 16, 16)
                        rows[k][row, sl] = rows[k][row, sl] * ws
                return 0
            lax.fori_loop(0, CHUNK // 16, _scale, 0)

            _scatter(c, k).start(add=True)
            _degsc(c).start(add=True)

            kp = (k - 1) % NBUF
            @pl.when(jnp.logical_and(c >= 1, c + NBUF - 1 < CPT))
            def _():
                _scatter(c - 1, kp).wait()
                _degsc(c - 1).wait()
                _gather(c + NBUF - 1, kp).start()
        return 0
    lax.fori_loop(0, CPT // NBUF, _iter, 0)

    for k3 in range(NBUF):
        c = CPT - NBUF + k3
        _scatter(c, c % NBUF).wait()
        _degsc(c).wait()

    plsc.subcore_barrier()

    @pl.when(cid == 0)
    def _():
        pltpu.sync_copy(agg_sh.at[pl.ds(base_r, RPT)],
                        agg0_out.at[pl.ds(base_r, RPT)])
        pltpu.sync_copy(deg_sh.at[pl.ds(base_r, RPT)],
                        deg0_out.at[pl.ds(base_r, RPT)])

    @pl.when(cid == 1)
    def _():
        pltpu.sync_copy(agg_sh.at[pl.ds(base_r, RPT)],
                        agg1_out.at[pl.ds(base_r, RPT)])
        pltpu.sync_copy(deg_sh.at[pl.ds(base_r, RPT)],
                        deg1_out.at[pl.ds(base_r, RPT)])


@functools.lru_cache(maxsize=1)
def _make_sc_call():
    return functools.partial(
        pl.kernel,
        out_type=[
            jax.ShapeDtypeStruct((NPAD, D), jnp.float32),
            jax.ShapeDtypeStruct((NPAD, D), jnp.float32),
            jax.ShapeDtypeStruct((NPAD,), jnp.float32),
            jax.ShapeDtypeStruct((NPAD,), jnp.float32),
        ],
        mesh=plsc.VectorSubcoreMesh(core_axis_name="c", subcore_axis_name="s",
                                    num_cores=2, num_subcores=16),
        compiler_params=pltpu.CompilerParams(use_tc_tiling_on_sc=False),
        scratch_types=[
            pltpu.VMEM_SHARED((NPAD, D), jnp.float32),
            pltpu.VMEM_SHARED((NPAD,), jnp.float32),
            pltpu.VMEM((CPT, CHUNK), jnp.int32),
            pltpu.VMEM((CPT, CHUNK), jnp.int32),
            pltpu.VMEM((CPT, CHUNK), jnp.float32),
            pltpu.VMEM((CHUNK, D), jnp.float32),
            pltpu.VMEM((CHUNK, D), jnp.float32),
            pltpu.VMEM((CHUNK, D), jnp.float32),
            pltpu.VMEM((CHUNK, D), jnp.float32),
            pltpu.VMEM((RPT // 10, D), jnp.float32),
            pltpu.VMEM((RPT,), jnp.float32),
            pltpu.SemaphoreType.DMA,
            pltpu.SemaphoreType.DMA,
            pltpu.SemaphoreType.DMA,
            pltpu.SemaphoreType.DMA,
            pltpu.SemaphoreType.DMA,
            pltpu.SemaphoreType.DMA,
            pltpu.SemaphoreType.DMA,
            pltpu.SemaphoreType.DMA,
            pltpu.SemaphoreType.DMA,
        ],
    )(_sc_body)


def _sc_call(t1, src2, dst2, w2):
    return _make_sc_call()(t1, src2, dst2, w2)


def _post_body(xe_ref, a0, a1, d0, d1,
               wc, bc, wp2, bp2, wq2, bq2, wr2, br2,
               g1, b1, g2, b2, wf1, bf1, wf2, bf2,
               wfs, bfs, wfgq, wfgo, bfg, gb1, bb1, wo, bo,
               out_ref):
    xe = xe_ref[...]
    agg = (a0[...] + a1[...]) * (1.0 / (d0[...] + d1[...] + 1e-5))
    t2 = jnp.maximum(jnp.dot(agg, wc[...], preferred_element_type=jnp.float32) + bc[...], 0.0)
    p2 = jnp.dot(t2, wp2[...], preferred_element_type=jnp.float32) + bp2[...]
    q2 = jnp.dot(t2, wq2[...], preferred_element_type=jnp.float32) + bq2[...]
    r2 = jnp.dot(t2, wr2[...], preferred_element_type=jnp.float32) + br2[...]
    o = jnp.maximum(p2 * _sigmoid(q2) + r2, 0.0)
    xn = _ln(xe, g1[...], b1[...])
    ff = jnp.maximum(jnp.dot(xn, wf1[...], preferred_element_type=jnp.float32) + bf1[...], 0.0)
    ff = jnp.dot(ff, wf2[...], preferred_element_type=jnp.float32) + bf2[...]
    us = _ln(ff + xn, g2[...], b2[...])
    fgx = (jnp.dot(xe, wfgq[...], preferred_element_type=jnp.float32)
           + jnp.dot(o, wfgo[...], preferred_element_type=jnp.float32) + bfg[...])
    g = _sigmoid(jnp.dot(us, wfs[...], preferred_element_type=jnp.float32) + bfs[...] + fgx)
    st = g * us + (1.0 - g) * fgx
    x1 = _ln(st + xe, gb1[...], bb1[...])
    out_ref[...] = jnp.dot(x1, wo[...], preferred_element_type=jnp.float32) + bo[...]


def _post_call(xe, a0, a1, d0, d1, *ws):
    full = lambda a: pl.BlockSpec(a.shape, lambda i: (0,) * a.ndim)
    blk = lambda: pl.BlockSpec((BLK, D), lambda i: (i, 0))
    return pl.pallas_call(
        _post_body,
        grid=(GRID,),
        in_specs=[blk(), blk(), blk(),
                  pl.BlockSpec((BLK, 1), lambda i: (i, 0)),
                  pl.BlockSpec((BLK, 1), lambda i: (i, 0))]
                 + [full(w) for w in ws],
        out_specs=pl.BlockSpec((BLK, OUT_LEN), lambda i: (i, 0)),
        out_shape=jax.ShapeDtypeStruct((N, OUT_LEN), jnp.float32),
    )(xe, a0, a1, d0, d1, *ws)


def kernel(x, edge_index, edge_weight, params):
    p = params
    r1 = lambda v: v.reshape(1, -1)
    npad = EPAD - E
    idx_pad = (jnp.arange(npad, dtype=jnp.int32) * 13) % N
    src2 = jnp.concatenate([edge_index[0], idx_pad]).reshape(EROWS, CHUNK)
    dst2 = jnp.concatenate([edge_index[1], idx_pad]).reshape(EROWS, CHUNK)
    w2 = jnp.concatenate([edge_weight,
                          jnp.zeros((npad,), jnp.float32)]).reshape(EROWS, CHUNK)

    xe, t1 = _pre_call(x, p['W_embed'], r1(p['b_embed']),
                       p['Wp1'], r1(p['bp1']), p['Wq1'], r1(p['bq1']),
                       p['Wr1'], r1(p['br1']))

    a0, a1, d0, d1 = _sc_call(t1, src2, dst2, w2)
    d0 = d0.reshape(NPAD, 1)
    d1 = d1.reshape(NPAD, 1)

    return _post_call(
        xe, a0, a1, d0, d1,
        p['Wc'], r1(p['bc']),
        p['Wp2'], r1(p['bp2']), p['Wq2'], r1(p['bq2']), p['Wr2'], r1(p['br2']),
        r1(p['g1']), r1(p['b1']), r1(p['g2']), r1(p['b2']),
        p['Wf1'], r1(p['bf1']), p['Wf2'], r1(p['bf2']),
        p['Wfs'], r1(p['bfs']),
        p['Wfg'][:D], p['Wfg'][D:], r1(p['bfg']),
        r1(p['gb1']), r1(p['bb1']),
        p['W_out'], r1(p['b_out']),
    )


# speedup vs baseline: 26.1022x; 1.7042x over previous
import functools

import jax
import jax.numpy as jnp
from jax import lax
from jax.experimental import pallas as pl
from jax.experimental.pallas import tpu as pltpu
from jax.experimental.pallas import tpu_sc as plsc

N = 10000
E = 320000
D_IN = 128
D = 64
OUT_LEN = 12
FE = 4

NPAD = 10240
RPT = 640
CHUNK = 128
EPAD = 327680
EROWS = EPAD // CHUNK
NW = 32
CPT = EROWS // NW
NBUF = 4
BLK = 1000
GRID = N // BLK


def _sigmoid(x):
    return 1.0 / (1.0 + jnp.exp(-x))


def _ln(x, g, b):
    m = jnp.mean(x, axis=-1, keepdims=True)
    v = jnp.mean((x - m) ** 2, axis=-1, keepdims=True)
    return (x - m) / jnp.sqrt(v + 1e-5) * g + b


def _pre_body(x_ref, we, be, wp, bp, wq, bq, wr, br, xe_ref, t1_ref):
    xe = jnp.dot(x_ref[...], we[...], preferred_element_type=jnp.float32) + be[...]
    p = jnp.dot(xe, wp[...], preferred_element_type=jnp.float32) + bp[...]
    q = jnp.dot(xe, wq[...], preferred_element_type=jnp.float32) + bq[...]
    r = jnp.dot(xe, wr[...], preferred_element_type=jnp.float32) + br[...]
    xe_ref[...] = xe
    t1_ref[...] = jnp.maximum(p * _sigmoid(q) + r, 0.0)


def _pre_call(x, we, be, wp, bp, wq, bq, wr, br):
    full = lambda s: pl.BlockSpec(s, lambda i: (0, 0))
    return pl.pallas_call(
        _pre_body,
        grid=(GRID,),
        in_specs=[
            pl.BlockSpec((BLK, D_IN), lambda i: (i, 0)),
            full((D_IN, D)), full((1, D)),
            full((D, D)), full((1, D)),
            full((D, D)), full((1, D)),
            full((D, D)), full((1, D)),
        ],
        out_specs=[
            pl.BlockSpec((BLK, D), lambda i: (i, 0)),
            pl.BlockSpec((BLK, D), lambda i: (i, 0)),
        ],
        out_shape=[
            jax.ShapeDtypeStruct((N, D), jnp.float32),
            jax.ShapeDtypeStruct((N, D), jnp.float32),
        ],
    )(x, we, be, wp, bp, wq, bq, wr, br)


def _sc_body(t1_hbm, src_hbm, dst_hbm, w_hbm, agg0_out, agg1_out,
             deg0_out, deg1_out,
             agg_sh, deg_sh, src_all, dst_all, w_all,
             r0, r1, r2, r3, sb0, sb1, zb, zd,
             g0, g1, g2, g3, s0, s1, s2, s3, dsem):
    cid = lax.axis_index("c")
    sid = lax.axis_index("s")
    wid = cid * 16 + sid
    zero16 = jnp.zeros((16,), jnp.float32)
    base_r = sid * RPT
    rows = (r0, r1, r2, r3)
    sbufs = (sb0, sb1)
    gsems = (g0, g1, g2, g3)
    ssems = (s0, s1, s2, s3)

    def _zrow(i, _):
        for j in range(4):
            zb[i, pl.ds(j * 16, 16)] = zero16
        return 0
    lax.fori_loop(0, RPT // 10, _zrow, 0)

    def _zdl(i, _):
        zd[pl.ds(i * 16, 16)] = zero16
        return 0
    lax.fori_loop(0, RPT // 16, _zdl, 0)

    for part in range(10):
        pltpu.sync_copy(zb, agg_sh.at[pl.ds(base_r + part * (RPT // 10), RPT // 10)])
    pltpu.sync_copy(zd, deg_sh.at[pl.ds(base_r, RPT)])

    ebase = wid * CPT
    pltpu.sync_copy(src_hbm.at[pl.ds(ebase, CPT)], src_all)
    pltpu.sync_copy(dst_hbm.at[pl.ds(ebase, CPT)], dst_all)
    pltpu.sync_copy(w_hbm.at[pl.ds(ebase, CPT)], w_all)

    plsc.subcore_barrier()

    def _gather(c, k):
        return pltpu.make_async_copy(t1_hbm.at[src_all.at[c]], rows[k], gsems[k])

    def _scatter(c, m):
        return pltpu.make_async_copy(sbufs[m], agg_sh.at[dst_all.at[c]], ssems[m])

    def _degsc(c):
        return pltpu.make_async_copy(w_all.at[c], deg_sh.at[dst_all.at[c]], dsem)

    for k in range(NBUF):
        _gather(k, k).start()

    def _iter(p, _):
        for k in range(NBUF):
            c = p * NBUF + k
            m = k % 2
            @pl.when(c >= 2)
            def _():
                _scatter(c - 2, m).wait()
                _degsc(c - 2).wait()
            _gather(c, k).wait()

            @plsc.parallel_loop(0, CHUNK // 16, unroll=2)
            def _scale(q):
                wv = w_all[c, pl.ds(q * 16, 16)]
                for j in range(16):
                    ws = wv[j]
                    row = q * 16 + j
                    for t in range(4):
                        sl = pl.ds(t * 16, 16)
                        sbufs[m][row, sl] = rows[k][row, sl] * ws

            _scatter(c, m).start(add=True)
            _degsc(c).start(add=True)

            @pl.when(c + NBUF < CPT)
            def _():
                _gather(c + NBUF, k).start()
        return 0
    lax.fori_loop(0, CPT // NBUF, _iter, 0)

    for k3 in range(2):
        c = CPT - 2 + k3
        _scatter(c, c % 2).wait()
        _degsc(c).wait()

    plsc.subcore_barrier()

    @pl.when(cid == 0)
    def _():
        pltpu.sync_copy(agg_sh.at[pl.ds(base_r, RPT)],
                        agg0_out.at[pl.ds(base_r, RPT)])
        pltpu.sync_copy(deg_sh.at[pl.ds(base_r, RPT)],
                        deg0_out.at[pl.ds(base_r, RPT)])

    @pl.when(cid == 1)
    def _():
        pltpu.sync_copy(agg_sh.at[pl.ds(base_r, RPT)],
                        agg1_out.at[pl.ds(base_r, RPT)])
        pltpu.sync_copy(deg_sh.at[pl.ds(base_r, RPT)],
                        deg1_out.at[pl.ds(base_r, RPT)])


@functools.lru_cache(maxsize=1)
def _make_sc_call():
    return functools.partial(
        pl.kernel,
        out_type=[
            jax.ShapeDtypeStruct((NPAD, D), jnp.float32),
            jax.ShapeDtypeStruct((NPAD, D), jnp.float32),
            jax.ShapeDtypeStruct((NPAD,), jnp.float32),
            jax.ShapeDtypeStruct((NPAD,), jnp.float32),
        ],
        mesh=plsc.VectorSubcoreMesh(core_axis_name="c", subcore_axis_name="s",
                                    num_cores=2, num_subcores=16),
        compiler_params=pltpu.CompilerParams(use_tc_tiling_on_sc=False),
        scratch_types=[
            pltpu.VMEM_SHARED((NPAD, D), jnp.float32),
            pltpu.VMEM_SHARED((NPAD,), jnp.float32),
            pltpu.VMEM((CPT, CHUNK), jnp.int32),
            pltpu.VMEM((CPT, CHUNK), jnp.int32),
            pltpu.VMEM((CPT, CHUNK), jnp.float32),
            pltpu.VMEM((CHUNK, D), jnp.float32),
            pltpu.VMEM((CHUNK, D), jnp.float32),
            pltpu.VMEM((CHUNK, D), jnp.float32),
            pltpu.VMEM((CHUNK, D), jnp.float32),
            pltpu.VMEM((CHUNK, D), jnp.float32),
            pltpu.VMEM((CHUNK, D), jnp.float32),
            pltpu.VMEM((RPT // 10, D), jnp.float32),
            pltpu.VMEM((RPT,), jnp.float32),
            pltpu.SemaphoreType.DMA,
            pltpu.SemaphoreType.DMA,
            pltpu.SemaphoreType.DMA,
            pltpu.SemaphoreType.DMA,
            pltpu.SemaphoreType.DMA,
            pltpu.SemaphoreType.DMA,
            pltpu.SemaphoreType.DMA,
            pltpu.SemaphoreType.DMA,
            pltpu.SemaphoreType.DMA,
        ],
    )(_sc_body)


def _sc_call(t1, src2, dst2, w2):
    return _make_sc_call()(t1, src2, dst2, w2)


def _post_body(xe_ref, a0, a1, d0, d1,
               wc, bc, wp2, bp2, wq2, bq2, wr2, br2,
               g1, b1, g2, b2, wf1, bf1, wf2, bf2,
               wfs, bfs, wfgq, wfgo, bfg, gb1, bb1, wo, bo,
               out_ref):
    xe = xe_ref[...]
    agg = (a0[...] + a1[...]) * (1.0 / (d0[...] + d1[...] + 1e-5))
    t2 = jnp.maximum(jnp.dot(agg, wc[...], preferred_element_type=jnp.float32) + bc[...], 0.0)
    p2 = jnp.dot(t2, wp2[...], preferred_element_type=jnp.float32) + bp2[...]
    q2 = jnp.dot(t2, wq2[...], preferred_element_type=jnp.float32) + bq2[...]
    r2 = jnp.dot(t2, wr2[...], preferred_element_type=jnp.float32) + br2[...]
    o = jnp.maximum(p2 * _sigmoid(q2) + r2, 0.0)
    xn = _ln(xe, g1[...], b1[...])
    ff = jnp.maximum(jnp.dot(xn, wf1[...], preferred_element_type=jnp.float32) + bf1[...], 0.0)
    ff = jnp.dot(ff, wf2[...], preferred_element_type=jnp.float32) + bf2[...]
    us = _ln(ff + xn, g2[...], b2[...])
    fgx = (jnp.dot(xe, wfgq[...], preferred_element_type=jnp.float32)
           + jnp.dot(o, wfgo[...], preferred_element_type=jnp.float32) + bfg[...])
    g = _sigmoid(jnp.dot(us, wfs[...], preferred_element_type=jnp.float32) + bfs[...] + fgx)
    st = g * us + (1.0 - g) * fgx
    x1 = _ln(st + xe, gb1[...], bb1[...])
    out_ref[...] = jnp.dot(x1, wo[...], preferred_element_type=jnp.float32) + bo[...]


def _post_call(xe, a0, a1, d0, d1, *ws):
    full = lambda a: pl.BlockSpec(a.shape, lambda i: (0,) * a.ndim)
    blk = lambda: pl.BlockSpec((BLK, D), lambda i: (i, 0))
    return pl.pallas_call(
        _post_body,
        grid=(GRID,),
        in_specs=[blk(), blk(), blk(),
                  pl.BlockSpec((BLK, 1), lambda i: (i, 0)),
                  pl.BlockSpec((BLK, 1), lambda i: (i, 0))]
                 + [full(w) for w in ws],
        out_specs=pl.BlockSpec((BLK, OUT_LEN), lambda i: (i, 0)),
        out_shape=jax.ShapeDtypeStruct((N, OUT_LEN), jnp.float32),
    )(xe, a0, a1, d0, d1, *ws)


def kernel(x, edge_index, edge_weight, params):
    p = params
    r1 = lambda v: v.reshape(1, -1)
    npad = EPAD - E
    idx_pad = (jnp.arange(npad, dtype=jnp.int32) * 13) % N
    src2 = jnp.concatenate([edge_index[0], idx_pad]).reshape(EROWS, CHUNK)
    dst2 = jnp.concatenate([edge_index[1], idx_pad]).reshape(EROWS, CHUNK)
    w2 = jnp.concatenate([edge_weight,
                          jnp.zeros((npad,), jnp.float32)]).reshape(EROWS, CHUNK)

    xe, t1 = _pre_call(x, p['W_embed'], r1(p['b_embed']),
                       p['Wp1'], r1(p['bp1']), p['Wq1'], r1(p['bq1']),
                       p['Wr1'], r1(p['br1']))

    a0, a1, d0, d1 = _sc_call(t1, src2, dst2, w2)
    d0 = d0.reshape(NPAD, 1)
    d1 = d1.reshape(NPAD, 1)

    return _post_call(
        xe, a0, a1, d0, d1,
        p['Wc'], r1(p['bc']),
        p['Wp2'], r1(p['bp2']), p['Wq2'], r1(p['bq2']), p['Wr2'], r1(p['br2']),
        r1(p['g1']), r1(p['b1']), r1(p['g2']), r1(p['b2']),
        p['Wf1'], r1(p['bf1']), p['Wf2'], r1(p['bf2']),
        p['Wfs'], r1(p['bfs']),
        p['Wfg'][:D], p['Wfg'][D:], r1(p['bfg']),
        r1(p['gb1']), r1(p['bb1']),
        p['W_out'], r1(p['b_out']),
    )


# speedup vs baseline: 26.1077x; 1.0002x over previous
import functools

import numpy as np
import jax
import jax.numpy as jnp
from jax import lax
from jax.experimental import pallas as pl
from jax.experimental.pallas import tpu as pltpu
from jax.experimental.pallas import tpu_sc as plsc

N = 10000
E = 320000
D_IN = 128
D = 64
OUT_LEN = 12
FE = 4

NPAD = 10240
RPT = 640
CHUNK = 128
EPAD = 327680
EROWS = EPAD // CHUNK
NW = 32
CPT = EROWS // NW
NBUF = 4
BLK = 1000
GRID = N // BLK


def _sigmoid(x):
    return 1.0 / (1.0 + jnp.exp(-x))


def _ln(x, g, b):
    m = jnp.mean(x, axis=-1, keepdims=True)
    v = jnp.mean((x - m) ** 2, axis=-1, keepdims=True)
    return (x - m) / jnp.sqrt(v + 1e-5) * g + b


def _pre_body(x_ref, we, be, wp, bp, wq, bq, wr, br, xe_ref, t1_ref):
    xe = jnp.dot(x_ref[...], we[...], preferred_element_type=jnp.float32) + be[...]
    p = jnp.dot(xe, wp[...], preferred_element_type=jnp.float32) + bp[...]
    q = jnp.dot(xe, wq[...], preferred_element_type=jnp.float32) + bq[...]
    r = jnp.dot(xe, wr[...], preferred_element_type=jnp.float32) + br[...]
    xe_ref[...] = xe
    t1_ref[...] = jnp.maximum(p * _sigmoid(q) + r, 0.0)


def _pre_call(x, we, be, wp, bp, wq, bq, wr, br):
    full = lambda s: pl.BlockSpec(s, lambda i: (0, 0))
    return pl.pallas_call(
        _pre_body,
        grid=(GRID,),
        in_specs=[
            pl.BlockSpec((BLK, D_IN), lambda i: (i, 0)),
            full((D_IN, D)), full((1, D)),
            full((D, D)), full((1, D)),
            full((D, D)), full((1, D)),
            full((D, D)), full((1, D)),
        ],
        out_specs=[
            pl.BlockSpec((BLK, D), lambda i: (i, 0)),
            pl.BlockSpec((BLK, D), lambda i: (i, 0)),
        ],
        out_shape=[
            jax.ShapeDtypeStruct((N, D), jnp.float32),
            jax.ShapeDtypeStruct((N, D), jnp.float32),
        ],
    )(x, we, be, wp, bp, wq, bq, wr, br)


def _sc_body(t1_hbm, src_hbm, dst_hbm, w_hbm, agg0_out, agg1_out,
             deg0_out, deg1_out,
             agg_sh, deg_sh, src_all, dst_all, w_all,
             r0, r1, r2, r3, sb0, sb1, zb, zd,
             g0, g1, g2, g3, s0, s1, s2, s3, dsem):
    cid = lax.axis_index("c")
    sid = lax.axis_index("s")
    wid = cid * 16 + sid
    zero16 = jnp.zeros((16,), jnp.float32)
    base_r = sid * RPT
    rows = (r0, r1, r2, r3)
    sbufs = (sb0, sb1)
    gsems = (g0, g1, g2, g3)
    ssems = (s0, s1, s2, s3)

    def _zrow(i, _):
        for j in range(4):
            zb[i, pl.ds(j * 16, 16)] = zero16
        return 0
    lax.fori_loop(0, RPT // 10, _zrow, 0)

    def _zdl(i, _):
        zd[pl.ds(i * 16, 16)] = zero16
        return 0
    lax.fori_loop(0, RPT // 16, _zdl, 0)

    for part in range(10):
        pltpu.sync_copy(zb, agg_sh.at[pl.ds(base_r + part * (RPT // 10), RPT // 10)])
    pltpu.sync_copy(zd, deg_sh.at[pl.ds(base_r, RPT)])

    ebase = wid * CPT * CHUNK
    pltpu.sync_copy(src_hbm.at[pl.ds(ebase, CPT * CHUNK)], src_all)
    pltpu.sync_copy(dst_hbm.at[pl.ds(ebase, CPT * CHUNK)], dst_all)
    pltpu.sync_copy(w_hbm.at[pl.ds(ebase, CPT * CHUNK)], w_all)

    plsc.subcore_barrier()

    def _gather(c, k):
        return pltpu.make_async_copy(
            t1_hbm.at[src_all.at[pl.ds(c * CHUNK, CHUNK)]], rows[k], gsems[k])

    def _scatter(c, m):
        return pltpu.make_async_copy(
            sbufs[m], agg_sh.at[dst_all.at[pl.ds(c * CHUNK, CHUNK)]], ssems[m])

    def _degsc(c):
        return pltpu.make_async_copy(
            w_all.at[pl.ds(c * CHUNK, CHUNK)],
            deg_sh.at[dst_all.at[pl.ds(c * CHUNK, CHUNK)]], dsem)

    for k in range(NBUF):
        _gather(k, k).start()

    def _iter(p, _):
        for k in range(NBUF):
            c = p * NBUF + k
            m = k % 2
            @pl.when(c >= 2)
            def _():
                _scatter(c - 2, m).wait()
                _degsc(c - 2).wait()
            _gather(c, k).wait()

            @plsc.parallel_loop(0, CHUNK // 16, unroll=2)
            def _scale(q):
                wv = w_all[pl.ds(c * CHUNK + q * 16, 16)]
                for j in range(16):
                    ws = wv[j]
                    row = q * 16 + j
                    for t in range(4):
                        sl = pl.ds(t * 16, 16)
                        sbufs[m][row, sl] = rows[k][row, sl] * ws

            _scatter(c, m).start(add=True)
            _degsc(c).start(add=True)

            @pl.when(c + NBUF < CPT)
            def _():
                _gather(c + NBUF, k).start()
        return 0
    lax.fori_loop(0, CPT // NBUF, _iter, 0)

    for k3 in range(2):
        c = CPT - 2 + k3
        _scatter(c, c % 2).wait()
        _degsc(c).wait()

    plsc.subcore_barrier()

    @pl.when(cid == 0)
    def _():
        pltpu.sync_copy(agg_sh.at[pl.ds(base_r, RPT)],
                        agg0_out.at[pl.ds(base_r, RPT)])
        pltpu.sync_copy(deg_sh.at[pl.ds(base_r, RPT)],
                        deg0_out.at[pl.ds(base_r, RPT)])

    @pl.when(cid == 1)
    def _():
        pltpu.sync_copy(agg_sh.at[pl.ds(base_r, RPT)],
                        agg1_out.at[pl.ds(base_r, RPT)])
        pltpu.sync_copy(deg_sh.at[pl.ds(base_r, RPT)],
                        deg1_out.at[pl.ds(base_r, RPT)])


@functools.lru_cache(maxsize=1)
def _make_sc_call():
    return functools.partial(
        pl.kernel,
        out_type=[
            jax.ShapeDtypeStruct((NPAD, D), jnp.float32),
            jax.ShapeDtypeStruct((NPAD, D), jnp.float32),
            jax.ShapeDtypeStruct((NPAD,), jnp.float32),
            jax.ShapeDtypeStruct((NPAD,), jnp.float32),
        ],
        mesh=plsc.VectorSubcoreMesh(core_axis_name="c", subcore_axis_name="s",
                                    num_cores=2, num_subcores=16),
        compiler_params=pltpu.CompilerParams(use_tc_tiling_on_sc=False),
        scratch_types=[
            pltpu.VMEM_SHARED((NPAD, D), jnp.float32),
            pltpu.VMEM_SHARED((NPAD,), jnp.float32),
            pltpu.VMEM((CPT * CHUNK,), jnp.int32),
            pltpu.VMEM((CPT * CHUNK,), jnp.int32),
            pltpu.VMEM((CPT * CHUNK,), jnp.float32),
            pltpu.VMEM((CHUNK, D), jnp.float32),
            pltpu.VMEM((CHUNK, D), jnp.float32),
            pltpu.VMEM((CHUNK, D), jnp.float32),
            pltpu.VMEM((CHUNK, D), jnp.float32),
            pltpu.VMEM((CHUNK, D), jnp.float32),
            pltpu.VMEM((CHUNK, D), jnp.float32),
            pltpu.VMEM((RPT // 10, D), jnp.float32),
            pltpu.VMEM((RPT,), jnp.float32),
            pltpu.SemaphoreType.DMA,
            pltpu.SemaphoreType.DMA,
            pltpu.SemaphoreType.DMA,
            pltpu.SemaphoreType.DMA,
            pltpu.SemaphoreType.DMA,
            pltpu.SemaphoreType.DMA,
            pltpu.SemaphoreType.DMA,
            pltpu.SemaphoreType.DMA,
            pltpu.SemaphoreType.DMA,
        ],
    )(_sc_body)


def _sc_call(t1, src2, dst2, w2):
    return _make_sc_call()(t1, src2, dst2, w2)


def _post_body(xe_ref, a0, a1, d0, d1,
               wc, bc, wp2, bp2, wq2, bq2, wr2, br2,
               g1, b1, g2, b2, wf1, bf1, wf2, bf2,
               wfs, bfs, wfgq, wfgo, bfg, gb1, bb1, wo, bo,
               out_ref):
    xe = xe_ref[...]
    agg = (a0[...] + a1[...]) * (1.0 / (d0[...] + d1[...] + 1e-5))
    t2 = jnp.maximum(jnp.dot(agg, wc[...], preferred_element_type=jnp.float32) + bc[...], 0.0)
    p2 = jnp.dot(t2, wp2[...], preferred_element_type=jnp.float32) + bp2[...]
    q2 = jnp.dot(t2, wq2[...], preferred_element_type=jnp.float32) + bq2[...]
    r2 = jnp.dot(t2, wr2[...], preferred_element_type=jnp.float32) + br2[...]
    o = jnp.maximum(p2 * _sigmoid(q2) + r2, 0.0)
    xn = _ln(xe, g1[...], b1[...])
    ff = jnp.maximum(jnp.dot(xn, wf1[...], preferred_element_type=jnp.float32) + bf1[...], 0.0)
    ff = jnp.dot(ff, wf2[...], preferred_element_type=jnp.float32) + bf2[...]
    us = _ln(ff + xn, g2[...], b2[...])
    fgx = (jnp.dot(xe, wfgq[...], preferred_element_type=jnp.float32)
           + jnp.dot(o, wfgo[...], preferred_element_type=jnp.float32) + bfg[...])
    g = _sigmoid(jnp.dot(us, wfs[...], preferred_element_type=jnp.float32) + bfs[...] + fgx)
    st = g * us + (1.0 - g) * fgx
    x1 = _ln(st + xe, gb1[...], bb1[...])
    out_ref[...] = jnp.dot(x1, wo[...], preferred_element_type=jnp.float32) + bo[...]


def _post_call(xe, a0, a1, d0, d1, *ws):
    full = lambda a: pl.BlockSpec(a.shape, lambda i: (0,) * a.ndim)
    blk = lambda: pl.BlockSpec((BLK, D), lambda i: (i, 0))
    return pl.pallas_call(
        _post_body,
        grid=(GRID,),
        in_specs=[blk(), blk(), blk(),
                  pl.BlockSpec((BLK, 1), lambda i: (i, 0)),
                  pl.BlockSpec((BLK, 1), lambda i: (i, 0))]
                 + [full(w) for w in ws],
        out_specs=pl.BlockSpec((BLK, OUT_LEN), lambda i: (i, 0)),
        out_shape=jax.ShapeDtypeStruct((N, OUT_LEN), jnp.float32),
    )(xe, a0, a1, d0, d1, *ws)


def kernel(x, edge_index, edge_weight, params):
    p = params
    r1 = lambda v: v.reshape(1, -1)
    npad = EPAD - E
    idx_pad = jnp.asarray(np.arange(npad, dtype=np.int32) * 13 % N)
    w_pad = jnp.zeros((npad,), jnp.float32)
    src2 = jnp.concatenate([edge_index[0], idx_pad])
    dst2 = jnp.concatenate([edge_index[1], idx_pad])
    w2 = jnp.concatenate([edge_weight, w_pad])

    xe, t1 = _pre_call(x, p['W_embed'], r1(p['b_embed']),
                       p['Wp1'], r1(p['bp1']), p['Wq1'], r1(p['bq1']),
                       p['Wr1'], r1(p['br1']))

    a0, a1, d0, d1 = _sc_call(t1, src2, dst2, w2)
    d0 = d0.reshape(NPAD, 1)
    d1 = d1.reshape(NPAD, 1)

    return _post_call(
        xe, a0, a1, d0, d1,
        p['Wc'], r1(p['bc']),
        p['Wp2'], r1(p['bp2']), p['Wq2'], r1(p['bq2']), p['Wr2'], r1(p['br2']),
        r1(p['g1']), r1(p['b1']), r1(p['g2']), r1(p['b2']),
        p['Wf1'], r1(p['bf1']), p['Wf2'], r1(p['bf2']),
        p['Wfs'], r1(p['bfs']),
        p['Wfg'][:D], p['Wfg'][D:], r1(p['bfg']),
        r1(p['gb1']), r1(p['bb1']),
        p['W_out'], r1(p['b_out']),
    )


# speedup vs baseline: 26.4055x; 1.0114x over previous
import functools

import numpy as np
import jax
import jax.numpy as jnp
from jax import lax
from jax.experimental import pallas as pl
from jax.experimental.pallas import tpu as pltpu
from jax.experimental.pallas import tpu_sc as plsc

N = 10000
E = 320000
D_IN = 128
D = 64
OUT_LEN = 12
FE = 4

NPAD = 10240
RPT = 640
CHUNK = 128
EPAD = 327680
EROWS = EPAD // CHUNK
NW = 32
CPT = EROWS // NW
NBUF = 4
BLK = 1000
GRID = N // BLK


def _sigmoid(x):
    return 1.0 / (1.0 + jnp.exp(-x))


def _ln(x, g, b):
    m = jnp.mean(x, axis=-1, keepdims=True)
    v = jnp.mean((x - m) ** 2, axis=-1, keepdims=True)
    return (x - m) / jnp.sqrt(v + 1e-5) * g + b


def _pre_body(x_ref, we, be, wp, bp, wq, bq, wr, br, xe_ref, t1_ref):
    xe = jnp.dot(x_ref[...], we[...], preferred_element_type=jnp.float32) + be[...]
    p = jnp.dot(xe, wp[...], preferred_element_type=jnp.float32) + bp[...]
    q = jnp.dot(xe, wq[...], preferred_element_type=jnp.float32) + bq[...]
    r = jnp.dot(xe, wr[...], preferred_element_type=jnp.float32) + br[...]
    xe_ref[...] = xe
    t1_ref[...] = jnp.maximum(p * _sigmoid(q) + r, 0.0)


def _pre_call(x, we, be, wp, bp, wq, bq, wr, br):
    full = lambda s: pl.BlockSpec(s, lambda i: (0, 0))
    return pl.pallas_call(
        _pre_body,
        grid=(GRID,),
        in_specs=[
            pl.BlockSpec((BLK, D_IN), lambda i: (i, 0)),
            full((D_IN, D)), full((1, D)),
            full((D, D)), full((1, D)),
            full((D, D)), full((1, D)),
            full((D, D)), full((1, D)),
        ],
        out_specs=[
            pl.BlockSpec((BLK, D), lambda i: (i, 0)),
            pl.BlockSpec((BLK, D), lambda i: (i, 0)),
        ],
        out_shape=[
            jax.ShapeDtypeStruct((N, D), jnp.float32),
            jax.ShapeDtypeStruct((N, D), jnp.float32),
        ],
    )(x, we, be, wp, bp, wq, bq, wr, br)


def _sc_body(t1_hbm, src_hbm, dst_hbm, w_hbm, agg0_out, agg1_out,
             deg0_out, deg1_out,
             agg_sh, deg_sh, src_all, dst_all, w_all,
             r0, r1, r2, r3, sb0, sb1, zb, zd,
             g0, g1, g2, g3, s0, s1, s2, s3, dsem):
    cid = lax.axis_index("c")
    sid = lax.axis_index("s")
    wid = cid * 16 + sid
    zero16 = jnp.zeros((16,), jnp.float32)
    base_r = sid * RPT
    rows = (r0, r1, r2, r3)
    sbufs = (sb0, sb1)
    gsems = (g0, g1, g2, g3)
    ssems = (s0, s1, s2, s3)

    def _zrow(i, _):
        for j in range(4):
            zb[i, pl.ds(j * 16, 16)] = zero16
        return 0
    lax.fori_loop(0, RPT // 10, _zrow, 0)

    def _zdl(i, _):
        zd[pl.ds(i * 16, 16)] = zero16
        return 0
    lax.fori_loop(0, RPT // 16, _zdl, 0)

    for part in range(10):
        pltpu.sync_copy(zb, agg_sh.at[pl.ds(base_r + part * (RPT // 10), RPT // 10)])
    pltpu.sync_copy(zd, deg_sh.at[pl.ds(base_r, RPT)])

    ebase = wid * CPT * CHUNK
    pltpu.sync_copy(src_hbm.at[pl.ds(ebase, CPT * CHUNK)], src_all)
    pltpu.sync_copy(dst_hbm.at[pl.ds(ebase, CPT * CHUNK)], dst_all)
    pltpu.sync_copy(w_hbm.at[pl.ds(ebase, CPT * CHUNK)], w_all)

    plsc.subcore_barrier()

    def _gather(c, k):
        return pltpu.make_async_copy(
            t1_hbm.at[src_all.at[pl.ds(c * CHUNK, CHUNK)]], rows[k], gsems[k])

    def _scatter(c, m):
        return pltpu.make_async_copy(
            sbufs[m], agg_sh.at[dst_all.at[pl.ds(c * CHUNK, CHUNK)]], ssems[m])

    def _degsc(c):
        return pltpu.make_async_copy(
            w_all.at[pl.ds(c * CHUNK, CHUNK)],
            deg_sh.at[dst_all.at[pl.ds(c * CHUNK, CHUNK)]], dsem)

    for k in range(NBUF):
        _gather(k, k).start()

    def _iter(p, _):
        for k in range(NBUF):
            c = p * NBUF + k
            m = k % 2
            @pl.when(c >= 2)
            def _():
                _scatter(c - 2, m).wait()
                _degsc(c - 2).wait()
            _gather(c, k).wait()

            @plsc.parallel_loop(0, CHUNK // 16, unroll=2)
            def _scale(q):
                wv = w_all[pl.ds(c * CHUNK + q * 16, 16)]
                for j in range(16):
                    ws = wv[j]
                    row = q * 16 + j
                    for t in range(4):
                        sl = pl.ds(t * 16, 16)
                        sbufs[m][row, sl] = rows[k][row, sl] * ws

            _scatter(c, m).start(add=True)
            _degsc(c).start(add=True)

            @pl.when(c + NBUF < CPT)
            def _():
                _gather(c + NBUF, k).start()
        return 0
    lax.fori_loop(0, CPT // NBUF, _iter, 0)

    for k3 in range(2):
        c = CPT - 2 + k3
        _scatter(c, c % 2).wait()
        _degsc(c).wait()

    plsc.subcore_barrier()

    @pl.when(cid == 0)
    def _():
        pltpu.sync_copy(agg_sh.at[pl.ds(base_r, RPT)],
                        agg0_out.at[pl.ds(base_r, RPT)])
        pltpu.sync_copy(deg_sh.at[pl.ds(base_r, RPT)],
                        deg0_out.at[pl.ds(base_r, RPT)])

    @pl.when(cid == 1)
    def _():
        pltpu.sync_copy(agg_sh.at[pl.ds(base_r, RPT)],
                        agg1_out.at[pl.ds(base_r, RPT)])
        pltpu.sync_copy(deg_sh.at[pl.ds(base_r, RPT)],
                        deg1_out.at[pl.ds(base_r, RPT)])


@functools.lru_cache(maxsize=1)
def _make_sc_call():
    return functools.partial(
        pl.kernel,
        out_type=[
            jax.ShapeDtypeStruct((NPAD, D), jnp.float32),
            jax.ShapeDtypeStruct((NPAD, D), jnp.float32),
            jax.ShapeDtypeStruct((NPAD,), jnp.float32),
            jax.ShapeDtypeStruct((NPAD,), jnp.float32),
        ],
        mesh=plsc.VectorSubcoreMesh(core_axis_name="c", subcore_axis_name="s",
                                    num_cores=2, num_subcores=16),
        compiler_params=pltpu.CompilerParams(use_tc_tiling_on_sc=False),
        scratch_types=[
            pltpu.VMEM_SHARED((NPAD, D), jnp.float32),
            pltpu.VMEM_SHARED((NPAD,), jnp.float32),
            pltpu.VMEM((CPT * CHUNK,), jnp.int32),
            pltpu.VMEM((CPT * CHUNK,), jnp.int32),
            pltpu.VMEM((CPT * CHUNK,), jnp.float32),
            pltpu.VMEM((CHUNK, D), jnp.float32),
            pltpu.VMEM((CHUNK, D), jnp.float32),
            pltpu.VMEM((CHUNK, D), jnp.float32),
            pltpu.VMEM((CHUNK, D), jnp.float32),
            pltpu.VMEM((CHUNK, D), jnp.float32),
            pltpu.VMEM((CHUNK, D), jnp.float32),
            pltpu.VMEM((RPT // 10, D), jnp.float32),
            pltpu.VMEM((RPT,), jnp.float32),
            pltpu.SemaphoreType.DMA,
            pltpu.SemaphoreType.DMA,
            pltpu.SemaphoreType.DMA,
            pltpu.SemaphoreType.DMA,
            pltpu.SemaphoreType.DMA,
            pltpu.SemaphoreType.DMA,
            pltpu.SemaphoreType.DMA,
            pltpu.SemaphoreType.DMA,
            pltpu.SemaphoreType.DMA,
        ],
    )(_sc_body)


def _sc_call(t1, src2, dst2, w2):
    return _make_sc_call()(t1, src2, dst2, w2)


def _mid_body(xe_ref, g1, b1, g2, b2, wf1, bf1, wf2, bf2, wfgq,
              us_ref, fq_ref):
    xe = xe_ref[...]
    xn = _ln(xe, g1[...], b1[...])
    ff = jnp.maximum(jnp.dot(xn, wf1[...], preferred_element_type=jnp.float32) + bf1[...], 0.0)
    ff = jnp.dot(ff, wf2[...], preferred_element_type=jnp.float32) + bf2[...]
    us_ref[...] = _ln(ff + xn, g2[...], b2[...])
    fq_ref[...] = jnp.dot(xe, wfgq[...], preferred_element_type=jnp.float32)


def _mid_call(xe, *ws):
    full = lambda a: pl.BlockSpec(a.shape, lambda i: (0,) * a.ndim)
    return pl.pallas_call(
        _mid_body,
        grid=(GRID,),
        in_specs=[pl.BlockSpec((BLK, D), lambda i: (i, 0))] + [full(w) for w in ws],
        out_specs=[pl.BlockSpec((BLK, D), lambda i: (i, 0)),
                   pl.BlockSpec((BLK, D), lambda i: (i, 0))],
        out_shape=[jax.ShapeDtypeStruct((N, D), jnp.float32),
                   jax.ShapeDtypeStruct((N, D), jnp.float32)],
    )(xe, *ws)


def _post_body(xe_ref, a0, a1, d0, d1, us_ref, fq_ref,
               wc, bc, wp2, bp2, wq2, bq2, wr2, br2,
               wfs, bfs, wfgo, bfg, gb1, bb1, wo, bo,
               out_ref):
    xe = xe_ref[...]
    agg = (a0[...] + a1[...]) * (1.0 / (d0[...] + d1[...] + 1e-5))
    t2 = jnp.maximum(jnp.dot(agg, wc[...], preferred_element_type=jnp.float32) + bc[...], 0.0)
    p2 = jnp.dot(t2, wp2[...], preferred_element_type=jnp.float32) + bp2[...]
    q2 = jnp.dot(t2, wq2[...], preferred_element_type=jnp.float32) + bq2[...]
    r2 = jnp.dot(t2, wr2[...], preferred_element_type=jnp.float32) + br2[...]
    o = jnp.maximum(p2 * _sigmoid(q2) + r2, 0.0)
    us = us_ref[...]
    fgx = (fq_ref[...]
           + jnp.dot(o, wfgo[...], preferred_element_type=jnp.float32) + bfg[...])
    g = _sigmoid(jnp.dot(us, wfs[...], preferred_element_type=jnp.float32) + bfs[...] + fgx)
    st = g * us + (1.0 - g) * fgx
    x1 = _ln(st + xe, gb1[...], bb1[...])
    out_ref[...] = jnp.dot(x1, wo[...], preferred_element_type=jnp.float32) + bo[...]


def _post_call(xe, a0, a1, d0, d1, us, fq, *ws):
    full = lambda a: pl.BlockSpec(a.shape, lambda i: (0,) * a.ndim)
    blk = lambda: pl.BlockSpec((BLK, D), lambda i: (i, 0))
    return pl.pallas_call(
        _post_body,
        grid=(GRID,),
        in_specs=[blk(), blk(), blk(),
                  pl.BlockSpec((BLK, 1), lambda i: (i, 0)),
                  pl.BlockSpec((BLK, 1), lambda i: (i, 0)),
                  blk(), blk()]
                 + [full(w) for w in ws],
        out_specs=pl.BlockSpec((BLK, OUT_LEN), lambda i: (i, 0)),
        out_shape=jax.ShapeDtypeStruct((N, OUT_LEN), jnp.float32),
    )(xe, a0, a1, d0, d1, us, fq, *ws)


def kernel(x, edge_index, edge_weight, params):
    p = params
    r1 = lambda v: v.reshape(1, -1)
    npad = EPAD - E
    idx_pad = jnp.asarray(np.arange(npad, dtype=np.int32) * 13 % N)
    w_pad = jnp.zeros((npad,), jnp.float32)
    src2 = jnp.concatenate([edge_index[0], idx_pad])
    dst2 = jnp.concatenate([edge_index[1], idx_pad])
    w2 = jnp.concatenate([edge_weight, w_pad])

    xe, t1 = _pre_call(x, p['W_embed'], r1(p['b_embed']),
                       p['Wp1'], r1(p['bp1']), p['Wq1'], r1(p['bq1']),
                       p['Wr1'], r1(p['br1']))

    a0, a1, d0, d1 = _sc_call(t1, src2, dst2, w2)
    d0 = d0.reshape(NPAD, 1)
    d1 = d1.reshape(NPAD, 1)

    us, fq = _mid_call(xe,
                       r1(p['g1']), r1(p['b1']), r1(p['g2']), r1(p['b2']),
                       p['Wf1'], r1(p['bf1']), p['Wf2'], r1(p['bf2']),
                       p['Wfg'][:D])

    return _post_call(
        xe, a0, a1, d0, d1, us, fq,
        p['Wc'], r1(p['bc']),
        p['Wp2'], r1(p['bp2']), p['Wq2'], r1(p['bq2']), p['Wr2'], r1(p['br2']),
        p['Wfs'], r1(p['bfs']),
        p['Wfg'][D:], r1(p['bfg']),
        r1(p['gb1']), r1(p['bb1']),
        p['W_out'], r1(p['b_out']),
    )


# speedup vs baseline: 27.5600x; 1.0437x over previous
import functools

import numpy as np
import jax
import jax.numpy as jnp
from jax import lax
from jax.experimental import pallas as pl
from jax.experimental.pallas import tpu as pltpu
from jax.experimental.pallas import tpu_sc as plsc

N = 10000
E = 320000
D_IN = 128
D = 64
OUT_LEN = 12
FE = 4

NPAD = 10240
RPT = 640
CHUNK = 128
EPAD = 327680
EROWS = EPAD // CHUNK
NW = 32
CPT = EROWS // NW
NBUF = 4
BLK = 2000
GRID = N // BLK


def _sigmoid(x):
    return 1.0 / (1.0 + jnp.exp(-x))


def _ln(x, g, b):
    m = jnp.mean(x, axis=-1, keepdims=True)
    v = jnp.mean((x - m) ** 2, axis=-1, keepdims=True)
    return (x - m) / jnp.sqrt(v + 1e-5) * g + b


def _pre_body(x_ref, we, be, wp, bp, wq, bq, wr, br, xe_ref, t1_ref):
    xe = jnp.dot(x_ref[...], we[...], preferred_element_type=jnp.float32) + be[...]
    p = jnp.dot(xe, wp[...], preferred_element_type=jnp.float32) + bp[...]
    q = jnp.dot(xe, wq[...], preferred_element_type=jnp.float32) + bq[...]
    r = jnp.dot(xe, wr[...], preferred_element_type=jnp.float32) + br[...]
    xe_ref[...] = xe
    t1_ref[...] = jnp.maximum(p * _sigmoid(q) + r, 0.0)


def _pre_call(x, we, be, wp, bp, wq, bq, wr, br):
    full = lambda s: pl.BlockSpec(s, lambda i: (0, 0))
    return pl.pallas_call(
        _pre_body,
        grid=(GRID,),
        in_specs=[
            pl.BlockSpec((BLK, D_IN), lambda i: (i, 0)),
            full((D_IN, D)), full((1, D)),
            full((D, D)), full((1, D)),
            full((D, D)), full((1, D)),
            full((D, D)), full((1, D)),
        ],
        out_specs=[
            pl.BlockSpec((BLK, D), lambda i: (i, 0)),
            pl.BlockSpec((BLK, D), lambda i: (i, 0)),
        ],
        out_shape=[
            jax.ShapeDtypeStruct((N, D), jnp.float32),
            jax.ShapeDtypeStruct((N, D), jnp.float32),
        ],
    )(x, we, be, wp, bp, wq, bq, wr, br)


def _sc_body(t1_hbm, src_hbm, dst_hbm, w_hbm, agg0_out, agg1_out,
             deg0_out, deg1_out,
             agg_sh, deg_sh, src_all, dst_all, w_all,
             r0, r1, r2, r3, sb0, sb1, zb, zd,
             g0, g1, g2, g3, s0, s1, s2, s3, dsem):
    cid = lax.axis_index("c")
    sid = lax.axis_index("s")
    wid = cid * 16 + sid
    zero16 = jnp.zeros((16,), jnp.float32)
    base_r = sid * RPT
    rows = (r0, r1, r2, r3)
    sbufs = (sb0, sb1)
    gsems = (g0, g1, g2, g3)
    ssems = (s0, s1, s2, s3)

    def _zrow(i, _):
        for j in range(4):
            zb[i, pl.ds(j * 16, 16)] = zero16
        return 0
    lax.fori_loop(0, RPT // 10, _zrow, 0)

    def _zdl(i, _):
        zd[pl.ds(i * 16, 16)] = zero16
        return 0
    lax.fori_loop(0, RPT // 16, _zdl, 0)

    for part in range(10):
        pltpu.sync_copy(zb, agg_sh.at[pl.ds(base_r + part * (RPT // 10), RPT // 10)])
    pltpu.sync_copy(zd, deg_sh.at[pl.ds(base_r, RPT)])

    ebase = wid * CPT * CHUNK
    pltpu.sync_copy(src_hbm.at[pl.ds(ebase, CPT * CHUNK)], src_all)
    pltpu.sync_copy(dst_hbm.at[pl.ds(ebase, CPT * CHUNK)], dst_all)
    pltpu.sync_copy(w_hbm.at[pl.ds(ebase, CPT * CHUNK)], w_all)

    plsc.subcore_barrier()

    def _gather(c, k):
        return pltpu.make_async_copy(
            t1_hbm.at[src_all.at[pl.ds(c * CHUNK, CHUNK)]], rows[k], gsems[k])

    def _scatter(c, m):
        return pltpu.make_async_copy(
            sbufs[m], agg_sh.at[dst_all.at[pl.ds(c * CHUNK, CHUNK)]], ssems[m])

    def _degsc(c):
        return pltpu.make_async_copy(
            w_all.at[pl.ds(c * CHUNK, CHUNK)],
            deg_sh.at[dst_all.at[pl.ds(c * CHUNK, CHUNK)]], dsem)

    for k in range(NBUF):
        _gather(k, k).start()

    def _iter(p, _):
        for k in range(NBUF):
            c = p * NBUF + k
            m = k % 2
            @pl.when(c >= 2)
            def _():
                _scatter(c - 2, m).wait()
                _degsc(c - 2).wait()
            _gather(c, k).wait()

            @plsc.parallel_loop(0, CHUNK // 16, unroll=2)
            def _scale(q):
                wv = w_all[pl.ds(c * CHUNK + q * 16, 16)]
                for j in range(16):
                    ws = wv[j]
                    row = q * 16 + j
                    for t in range(4):
                        sl = pl.ds(t * 16, 16)
                        sbufs[m][row, sl] = rows[k][row, sl] * ws

            _scatter(c, m).start(add=True)
            _degsc(c).start(add=True)

            @pl.when(c + NBUF < CPT)
            def _():
                _gather(c + NBUF, k).start()
        return 0
    lax.fori_loop(0, CPT // NBUF, _iter, 0)

    for k3 in range(2):
        c = CPT - 2 + k3
        _scatter(c, c % 2).wait()
        _degsc(c).wait()

    plsc.subcore_barrier()

    @pl.when(cid == 0)
    def _():
        pltpu.sync_copy(agg_sh.at[pl.ds(base_r, RPT)],
                        agg0_out.at[pl.ds(base_r, RPT)])
        pltpu.sync_copy(deg_sh.at[pl.ds(base_r, RPT)],
                        deg0_out.at[pl.ds(base_r, RPT)])

    @pl.when(cid == 1)
    def _():
        pltpu.sync_copy(agg_sh.at[pl.ds(base_r, RPT)],
                        agg1_out.at[pl.ds(base_r, RPT)])
        pltpu.sync_copy(deg_sh.at[pl.ds(base_r, RPT)],
                        deg1_out.at[pl.ds(base_r, RPT)])


@functools.lru_cache(maxsize=1)
def _make_sc_call():
    return functools.partial(
        pl.kernel,
        out_type=[
            jax.ShapeDtypeStruct((NPAD, D), jnp.float32),
            jax.ShapeDtypeStruct((NPAD, D), jnp.float32),
            jax.ShapeDtypeStruct((NPAD,), jnp.float32),
            jax.ShapeDtypeStruct((NPAD,), jnp.float32),
        ],
        mesh=plsc.VectorSubcoreMesh(core_axis_name="c", subcore_axis_name="s",
                                    num_cores=2, num_subcores=16),
        compiler_params=pltpu.CompilerParams(use_tc_tiling_on_sc=False),
        scratch_types=[
            pltpu.VMEM_SHARED((NPAD, D), jnp.float32),
            pltpu.VMEM_SHARED((NPAD,), jnp.float32),
            pltpu.VMEM((CPT * CHUNK,), jnp.int32),
            pltpu.VMEM((CPT * CHUNK,), jnp.int32),
            pltpu.VMEM((CPT * CHUNK,), jnp.float32),
            pltpu.VMEM((CHUNK, D), jnp.float32),
            pltpu.VMEM((CHUNK, D), jnp.float32),
            pltpu.VMEM((CHUNK, D), jnp.float32),
            pltpu.VMEM((CHUNK, D), jnp.float32),
            pltpu.VMEM((CHUNK, D), jnp.float32),
            pltpu.VMEM((CHUNK, D), jnp.float32),
            pltpu.VMEM((RPT // 10, D), jnp.float32),
            pltpu.VMEM((RPT,), jnp.float32),
            pltpu.SemaphoreType.DMA,
            pltpu.SemaphoreType.DMA,
            pltpu.SemaphoreType.DMA,
            pltpu.SemaphoreType.DMA,
            pltpu.SemaphoreType.DMA,
            pltpu.SemaphoreType.DMA,
            pltpu.SemaphoreType.DMA,
            pltpu.SemaphoreType.DMA,
            pltpu.SemaphoreType.DMA,
        ],
    )(_sc_body)


def _sc_call(t1, src2, dst2, w2):
    return _make_sc_call()(t1, src2, dst2, w2)


def _mid_body(xe_ref, g1, b1, g2, b2, wf1, bf1, wf2, bf2, wfgq,
              us_ref, fq_ref):
    xe = xe_ref[...]
    xn = _ln(xe, g1[...], b1[...])
    ff = jnp.maximum(jnp.dot(xn, wf1[...], preferred_element_type=jnp.float32) + bf1[...], 0.0)
    ff = jnp.dot(ff, wf2[...], preferred_element_type=jnp.float32) + bf2[...]
    us_ref[...] = _ln(ff + xn, g2[...], b2[...])
    fq_ref[...] = jnp.dot(xe, wfgq[...], preferred_element_type=jnp.float32)


def _mid_call(xe, *ws):
    full = lambda a: pl.BlockSpec(a.shape, lambda i: (0,) * a.ndim)
    return pl.pallas_call(
        _mid_body,
        grid=(GRID,),
        in_specs=[pl.BlockSpec((BLK, D), lambda i: (i, 0))] + [full(w) for w in ws],
        out_specs=[pl.BlockSpec((BLK, D), lambda i: (i, 0)),
                   pl.BlockSpec((BLK, D), lambda i: (i, 0))],
        out_shape=[jax.ShapeDtypeStruct((N, D), jnp.float32),
                   jax.ShapeDtypeStruct((N, D), jnp.float32)],
    )(xe, *ws)


def _post_body(xe_ref, a0, a1, d0, d1, us_ref, fq_ref,
               wc, bc, wp2, bp2, wq2, bq2, wr2, br2,
               wfs, bfs, wfgo, bfg, gb1, bb1, wo, bo,
               out_ref):
    xe = xe_ref[...]
    agg = (a0[...] + a1[...]) * (1.0 / (d0[...] + d1[...] + 1e-5))
    t2 = jnp.maximum(jnp.dot(agg, wc[...], preferred_element_type=jnp.float32) + bc[...], 0.0)
    p2 = jnp.dot(t2, wp2[...], preferred_element_type=jnp.float32) + bp2[...]
    q2 = jnp.dot(t2, wq2[...], preferred_element_type=jnp.float32) + bq2[...]
    r2 = jnp.dot(t2, wr2[...], preferred_element_type=jnp.float32) + br2[...]
    o = jnp.maximum(p2 * _sigmoid(q2) + r2, 0.0)
    us = us_ref[...]
    fgx = (fq_ref[...]
           + jnp.dot(o, wfgo[...], preferred_element_type=jnp.float32) + bfg[...])
    g = _sigmoid(jnp.dot(us, wfs[...], preferred_element_type=jnp.float32) + bfs[...] + fgx)
    st = g * us + (1.0 - g) * fgx
    x1 = _ln(st + xe, gb1[...], bb1[...])
    out_ref[...] = jnp.dot(x1, wo[...], preferred_element_type=jnp.float32) + bo[...]


def _post_call(xe, a0, a1, d0, d1, us, fq, *ws):
    full = lambda a: pl.BlockSpec(a.shape, lambda i: (0,) * a.ndim)
    blk = lambda: pl.BlockSpec((BLK, D), lambda i: (i, 0))
    return pl.pallas_call(
        _post_body,
        grid=(GRID,),
        in_specs=[blk(), blk(), blk(),
                  pl.BlockSpec((BLK, 1), lambda i: (i, 0)),
                  pl.BlockSpec((BLK, 1), lambda i: (i, 0)),
                  blk(), blk()]
                 + [full(w) for w in ws],
        out_specs=pl.BlockSpec((BLK, OUT_LEN), lambda i: (i, 0)),
        out_shape=jax.ShapeDtypeStruct((N, OUT_LEN), jnp.float32),
    )(xe, a0, a1, d0, d1, us, fq, *ws)


def kernel(x, edge_index, edge_weight, params):
    p = params
    r1 = lambda v: v.reshape(1, -1)
    npad = EPAD - E
    idx_pad = jnp.asarray(np.arange(npad, dtype=np.int32) * 13 % N)
    w_pad = jnp.zeros((npad,), jnp.float32)
    src2 = jnp.concatenate([edge_index[0], idx_pad])
    dst2 = jnp.concatenate([edge_index[1], idx_pad])
    w2 = jnp.concatenate([edge_weight, w_pad])

    xe, t1 = _pre_call(x, p['W_embed'], r1(p['b_embed']),
                       p['Wp1'], r1(p['bp1']), p['Wq1'], r1(p['bq1']),
                       p['Wr1'], r1(p['br1']))

    a0, a1, d0, d1 = _sc_call(t1, src2, dst2, w2)
    d0 = d0.reshape(NPAD, 1)
    d1 = d1.reshape(NPAD, 1)

    us, fq = _mid_call(xe,
                       r1(p['g1']), r1(p['b1']), r1(p['g2']), r1(p['b2']),
                       p['Wf1'], r1(p['bf1']), p['Wf2'], r1(p['bf2']),
                       p['Wfg'][:D])

    return _post_call(
        xe, a0, a1, d0, d1, us, fq,
        p['Wc'], r1(p['bc']),
        p['Wp2'], r1(p['bp2']), p['Wq2'], r1(p['bq2']), p['Wr2'], r1(p['br2']),
        p['Wfs'], r1(p['bfs']),
        p['Wfg'][D:], r1(p['bfg']),
        r1(p['gb1']), r1(p['bb1']),
        p['W_out'], r1(p['b_out']),
    )
